# baseline (speedup 1.0000x reference)
import functools

import jax
import jax.numpy as jnp
from jax import lax
from jax.experimental import pallas as pl
from jax.experimental.pallas import tpu as pltpu
from jax.experimental.pallas import tpu_sc as plsc

EMBED = 128
VOCAB = 14
SEQ = 64
FUSED_ROWS = SEQ * VOCAB
NC = 2
NS = 16
NW = NC * NS
GCHUNK = 128
GPER = 2
CHUNK = GCHUNK * GPER
LANE = 16
ROWS_PER_TILE = FUSED_ROWS // NS
POS_PER_TILE = ROWS_PER_TILE // VOCAB


def _embed_sc(flat_idx, token_flat, pos_flat):
    n_rows = flat_idx.shape[0]
    rows_per_w = n_rows // NW
    n_chunks = rows_per_w // CHUNK
    mesh = plsc.VectorSubcoreMesh(core_axis_name="c", subcore_axis_name="s")

    @functools.partial(
        pl.kernel,
        out_type=jax.ShapeDtypeStruct((n_rows, EMBED), jnp.float32),
        mesh=mesh,
        scratch_types=[
            pltpu.VMEM((VOCAB * EMBED,), jnp.float32),
            pltpu.VMEM((POS_PER_TILE * EMBED,), jnp.float32),
            pltpu.VMEM((ROWS_PER_TILE, EMBED), jnp.float32),
            pltpu.VMEM((rows_per_w,), jnp.int32),
            pltpu.VMEM((CHUNK, EMBED), jnp.float32),
            pltpu.VMEM((CHUNK, EMBED), jnp.float32),
            pltpu.VMEM_SHARED((FUSED_ROWS, EMBED), jnp.float32),
            pltpu.SemaphoreType.DMA,
            pltpu.SemaphoreType.DMA,
            pltpu.SemaphoreType.DMA,
            pltpu.SemaphoreType.DMA,
        ],
    )
    def k(idx_hbm, tok_hbm, pos_hbm, out_hbm, tok_v, pos_v, fbuild, idx_all,
          rows_v0, rows_v1, fused_sp, sem_g0, sem_g1, sem_s0, sem_s1):
        sid = lax.axis_index("s")
        wid = sid * NC + lax.axis_index("c")
        base0 = wid * rows_per_w

        pltpu.sync_copy(idx_hbm.at[pl.ds(base0, rows_per_w)], idx_all)
        pltpu.sync_copy(tok_hbm, tok_v)
        pltpu.sync_copy(
            pos_hbm.at[pl.ds(sid * (POS_PER_TILE * EMBED), POS_PER_TILE * EMBED)],
            pos_v)

        for a in range(POS_PER_TILE):
            for v in range(VOCAB):
                j = a * VOCAB + v
                for g in range(EMBED // LANE):
                    sl = pl.ds(g * LANE, LANE)
                    fbuild[j, sl] = (tok_v[pl.ds(v * EMBED + g * LANE, LANE)]
                                     + pos_v[pl.ds(a * EMBED + g * LANE, LANE)])
        pltpu.sync_copy(fbuild, fused_sp.at[pl.ds(sid * ROWS_PER_TILE,
                                                  ROWS_PER_TILE)])

        lane = lax.iota(jnp.int32, 16)
        pats = [((lane + 16 * gg) % SEQ) * VOCAB for gg in range(4)]

        def transform(t, carry):
            for g in range(8):
                sl = pl.ds(t * (8 * LANE) + g * LANE, LANE)
                idx_all[sl] = idx_all[sl] + pats[g % 4]
            return carry

        lax.fori_loop(0, rows_per_w // (8 * LANE), transform, 0)
        plsc.subcore_barrier()

        bufs = ((rows_v0, sem_g0, sem_s0), (rows_v1, sem_g1, sem_s1))

        def body(j, carry):
            for b, (rows_v, sem_g, sem_s) in enumerate(bufs):
                i = j * 2 + b
                base = base0 + i * CHUNK

                @pl.when(j > 0)
                def _():
                    pltpu.make_async_copy(
                        rows_v, out_hbm.at[pl.ds(base0, CHUNK)], sem_s).wait()

                handles = [
                    pltpu.async_copy(
                        fused_sp.at[idx_all.at[
                            pl.ds(i * CHUNK + g * GCHUNK, GCHUNK)]],
                        rows_v.at[pl.ds(g * GCHUNK, GCHUNK)], sem_g)
                    for g in range(GPER)
                ]
                for h in handles:
                    h.wait()
                pltpu.async_copy(rows_v, out_hbm.at[pl.ds(base, CHUNK)], sem_s)
            return carry

        lax.fori_loop(0, n_chunks // 2, body, 0)
        for rows_v, _, sem_s in bufs:
            pltpu.make_async_copy(
                rows_v, out_hbm.at[pl.ds(base0, CHUNK)], sem_s).wait()

    return k(flat_idx, token_flat, pos_flat)


def kernel(inputs, token_table, pos_table):
    batch = inputs.shape[0]
    out2 = _embed_sc(inputs.reshape(batch * SEQ),
                     token_table.reshape(VOCAB * EMBED),
                     pos_table.reshape(SEQ * EMBED))
    return out2.reshape(batch, SEQ, EMBED)


# speedup vs baseline: 1.0679x; 1.0679x over previous
import functools

import jax
import jax.numpy as jnp
from jax import lax
from jax.experimental import pallas as pl
from jax.experimental.pallas import tpu as pltpu
from jax.experimental.pallas import tpu_sc as plsc

EMBED = 128
VOCAB = 14
SEQ = 64
FUSED_ROWS = SEQ * VOCAB
NC = 2
NS = 16
NW = NC * NS
GCHUNK = 128
GPER = 2
CHUNK = GCHUNK * GPER


def _prep_tc(flat, token_table, pos_table):
    b = flat.shape[0]

    def body(flat_ref, tok_ref, pos_ref, f_ref, idx_ref):
        f_ref[...] = pos_ref[...][:, None, :] + tok_ref[...][None, :, :]
        pos_ids = lax.broadcasted_iota(jnp.int32, (b, SEQ), 1)
        idx_ref[...] = flat_ref[...] + pos_ids * VOCAB

    return pl.pallas_call(
        body,
        out_shape=(
            jax.ShapeDtypeStruct((SEQ, VOCAB, EMBED), jnp.float32),
            jax.ShapeDtypeStruct((b, SEQ), jnp.int32),
        ),
    )(flat, token_table, pos_table)


def _gather_sc(fused, idx):
    n_rows = idx.shape[0]
    rows_per_w = n_rows // NW
    n_chunks = rows_per_w // CHUNK
    mesh = plsc.VectorSubcoreMesh(core_axis_name="c", subcore_axis_name="s")

    @functools.partial(
        pl.kernel,
        out_type=jax.ShapeDtypeStruct((n_rows, EMBED), jnp.float32),
        mesh=mesh,
        scratch_types=[
            pltpu.VMEM((rows_per_w,), jnp.int32),
            pltpu.VMEM((CHUNK, EMBED), jnp.float32),
            pltpu.VMEM((CHUNK, EMBED), jnp.float32),
            pltpu.VMEM_SHARED((FUSED_ROWS, EMBED), jnp.float32),
            pltpu.SemaphoreType.DMA,
            pltpu.SemaphoreType.DMA,
            pltpu.SemaphoreType.DMA,
            pltpu.SemaphoreType.DMA,
        ],
    )
    def k(fused_hbm, idx_hbm, out_hbm, idx_all, rows_v0, rows_v1, fused_sp,
          sem_g0, sem_g1, sem_s0, sem_s1):
        sid = lax.axis_index("s")
        wid = sid * NC + lax.axis_index("c")
        base0 = wid * rows_per_w

        @pl.when(sid == 0)
        def _():
            pltpu.sync_copy(fused_hbm, fused_sp)

        pltpu.sync_copy(idx_hbm.at[pl.ds(base0, rows_per_w)], idx_all)
        plsc.subcore_barrier()

        bufs = ((rows_v0, sem_g0, sem_s0), (rows_v1, sem_g1, sem_s1))

        def body(j, carry):
            for b, (rows_v, sem_g, sem_s) in enumerate(bufs):
                i = j * 2 + b
                base = base0 + i * CHUNK

                @pl.when(j > 0)
                def _():
                    pltpu.make_async_copy(
                        rows_v, out_hbm.at[pl.ds(base0, CHUNK)], sem_s).wait()

                handles = [
                    pltpu.async_copy(
                        fused_sp.at[idx_all.at[
                            pl.ds(i * CHUNK + g * GCHUNK, GCHUNK)]],
                        rows_v.at[pl.ds(g * GCHUNK, GCHUNK)], sem_g)
                    for g in range(GPER)
                ]
                for h in handles:
                    h.wait()
                pltpu.async_copy(rows_v, out_hbm.at[pl.ds(base, CHUNK)], sem_s)
            return carry

        lax.fori_loop(0, n_chunks // 2, body, 0)
        for rows_v, _, sem_s in bufs:
            pltpu.make_async_copy(
                rows_v, out_hbm.at[pl.ds(base0, CHUNK)], sem_s).wait()

    return k(fused, idx)


def kernel(inputs, token_table, pos_table):
    batch = inputs.shape[0]
    flat = inputs.reshape(batch, SEQ)
    fused3, idx = _prep_tc(flat, token_table, pos_table)
    out2 = _gather_sc(fused3.reshape(FUSED_ROWS, EMBED), idx.reshape(-1))
    return out2.reshape(batch, SEQ, EMBED)
